# baseline (speedup 1.0000x reference)
import functools

import jax
import jax.numpy as jnp
from jax import lax
from jax.experimental import pallas as pl
from jax.experimental.pallas import tpu as pltpu
from jax.experimental.pallas import tpu_sc as plsc

BATCH = 4096
NUM_NUMERIC = 13
NUM_CAT = 26
VOCAB = 100000
EMBED_DIM = 64

NC = 2
NS = 16
NW = NC * NS
B_PER_W = BATCH // NW
CHUNK_B = 4
ROWS_PER_CHUNK = CHUNK_B * NUM_CAT
N_CHUNKS = B_PER_W // CHUNK_B
L = 16
D_VECS = EMBED_DIM // L


@functools.partial(
    pl.kernel,
    out_type=jax.ShapeDtypeStruct((BATCH, L), jnp.float32),
    mesh=plsc.VectorSubcoreMesh(core_axis_name="c", subcore_axis_name="s"),
    compiler_params=pltpu.CompilerParams(use_tc_tiling_on_sc=False),
    scratch_types=[
        pltpu.VMEM((N_CHUNKS, ROWS_PER_CHUNK), jnp.int32),
        pltpu.VMEM((2, ROWS_PER_CHUNK, EMBED_DIM), jnp.float32),
        pltpu.VMEM((B_PER_W, L), jnp.float32),
        pltpu.SemaphoreType.DMA,
        pltpu.SemaphoreType.DMA,
    ],
)
def _fm_order2_sc(idx_hbm, table_hbm, out_hbm, idx_v, buf, out_v, sem0, sem1):
    wid = lax.axis_index("c") * NS + lax.axis_index("s")
    pltpu.sync_copy(idx_hbm.at[wid], idx_v)

    sems = (sem0, sem1)

    def start(j, b):
        pltpu.make_async_copy(
            table_hbm.at[idx_v.at[j]], buf.at[b], sems[b]
        ).start()

    def wait(b):
        pltpu.make_async_copy(
            table_hbm.at[idx_v.at[0]], buf.at[b], sems[b]
        ).wait()

    def compute(j, b):
        for i in range(CHUNK_B):
            acc = [jnp.zeros((L,), jnp.float32) for _ in range(D_VECS)]
            q = jnp.zeros((L,), jnp.float32)
            for f in range(NUM_CAT):
                r = i * NUM_CAT + f
                for dv in range(D_VECS):
                    x = buf[b, r, pl.ds(dv * L, L)]
                    acc[dv] = acc[dv] + x
                    q = q + x * x
            w = acc[0] * acc[0] + acc[1] * acc[1]
            w = w + acc[2] * acc[2] + acc[3] * acc[3]
            w = w - q
            out_v[j * CHUNK_B + i, :] = w

    start(0, 0)
    start(1, 1)

    def body(it, carry):
        j = it * 2
        for b in range(2):
            jj = j + b
            wait(b)
            compute(jj, b)

            @pl.when(jj + 2 < N_CHUNKS)
            def _():
                start(jj + 2, b)

        return carry

    lax.fori_loop(0, N_CHUNKS // 2, body, 0)

    pltpu.sync_copy(out_v, out_hbm.at[pl.ds(wid * B_PER_W, B_PER_W)])


def _head_tc(num_ref, cat_ref, wfm_ref, wn_ref, wc_ref, b_ref, out_ref):
    o1 = (
        jnp.sum(num_ref[:] * wn_ref[:], axis=1, keepdims=True)
        + jnp.sum(cat_ref[:] * wc_ref[:], axis=1, keepdims=True)
        + b_ref[0, 0]
    )
    o2 = 0.5 * jnp.sum(wfm_ref[:], axis=1, keepdims=True)
    z = o1 + o2
    out_ref[:] = 1.0 / (1.0 + jnp.exp(-z))


def kernel(numeric_features, cat_features, lin_w, lin_b, emb_tables):
    cat = cat_features.astype(jnp.int32)
    flat_idx = cat + (jnp.arange(NUM_CAT, dtype=jnp.int32) * VOCAB)[None, :]
    idx3 = flat_idx.reshape(NW, N_CHUNKS, ROWS_PER_CHUNK)
    table = emb_tables.reshape(NUM_CAT * VOCAB, EMBED_DIM)

    wfm = _fm_order2_sc(idx3, table)

    catf = cat_features.astype(jnp.float32)
    wn = lin_w[:, :NUM_NUMERIC]
    wc = lin_w[:, NUM_NUMERIC:]
    b2 = lin_b.reshape(1, 1)

    yhat = pl.pallas_call(
        _head_tc,
        out_shape=jax.ShapeDtypeStruct((BATCH, 1), jnp.float32),
    )(numeric_features, catf, wfm, wn, wc, b2)
    return yhat.reshape(-1)
